# baseline (speedup 1.0000x reference)
import functools

import jax
import jax.numpy as jnp
from jax import lax
from jax.experimental import pallas as pl
from jax.experimental.pallas import tpu as pltpu
from jax.experimental.pallas import tpu_sc as plsc

V = 100000
D = 64
F = 128
B = 4096
L = 50
TD = 2 * D

_NC, _NS = 2, 16
_NW = _NC * _NS
_SESS_PER_W = B // _NW
_CH = 4
_CHUNKS = _SESS_PER_W // _CH
_LANES = 16
_NSEG = TD // _LANES


_ROW_BLK = 8192


def _table_body(embT_ref, feats_ref, w_ref, b_ref, out_ref):
    out_ref[:, :D] = embT_ref[...].T
    mm = jnp.dot(feats_ref[...], w_ref[...],
                 preferred_element_type=jnp.float32)
    out_ref[:, D:] = mm + b_ref[...]


def _build_table(item_emb_table, item_features, feat_W, feat_b):
    n_rows = item_emb_table.shape[0]
    grid = (n_rows + _ROW_BLK - 1) // _ROW_BLK
    return pl.pallas_call(
        _table_body,
        grid=(grid,),
        in_specs=[
            pl.BlockSpec((D, _ROW_BLK), lambda i: (0, i)),
            pl.BlockSpec((_ROW_BLK, F), lambda i: (i, 0)),
            pl.BlockSpec((F, D), lambda i: (0, 0)),
            pl.BlockSpec((1, D), lambda i: (0, 0)),
        ],
        out_specs=pl.BlockSpec((_ROW_BLK, TD), lambda i: (i, 0)),
        out_shape=jax.ShapeDtypeStruct((n_rows, TD), jnp.float32),
    )(item_emb_table.T, item_features, feat_W, feat_b.reshape(1, D))


def _gather_sum(table, idx):
    mesh = plsc.VectorSubcoreMesh(core_axis_name="c", subcore_axis_name="s")

    @functools.partial(
        pl.kernel,
        out_type=jax.ShapeDtypeStruct((B, TD), jnp.float32),
        mesh=mesh,
        scratch_types=[
            pltpu.VMEM((_SESS_PER_W, L), jnp.int32),
            pltpu.VMEM((_CH, L, TD), jnp.float32),
            pltpu.VMEM((_CH, L, TD), jnp.float32),
            pltpu.VMEM((_SESS_PER_W, TD), jnp.float32),
            pltpu.SemaphoreType.DMA,
            pltpu.SemaphoreType.DMA,
        ],
    )
    def k(table_hbm, idx_hbm, out_hbm, idx_all, rows0, rows1, out_v,
          sem0, sem1):
        wid = lax.axis_index("s") * _NC + lax.axis_index("c")
        sess0 = wid * _SESS_PER_W
        pltpu.sync_copy(idx_hbm.at[pl.ds(sess0, _SESS_PER_W)], idx_all)

        def start(j, rows_v, sem):
            for s in range(_CH):
                pltpu.async_copy(
                    table_hbm.at[idx_all.at[j * _CH + s]], rows_v.at[s], sem)

        def drain(j, rows_v, sem):
            for s in range(_CH):
                pltpu.make_async_copy(
                    table_hbm.at[idx_all.at[j * _CH + s]], rows_v.at[s],
                    sem).wait()

        def reduce(j, rows_v):
            for s in range(_CH):
                accs = tuple(jnp.zeros((_LANES,), jnp.float32)
                             for _ in range(_NSEG))
                for l in range(L):
                    accs = tuple(
                        acc + rows_v[s, l, pl.ds(c * _LANES, _LANES)]
                        for c, acc in enumerate(accs)
                    )
                for c in range(_NSEG):
                    out_v[j * _CH + s, pl.ds(c * _LANES, _LANES)] = accs[c]

        start(0, rows0, sem0)

        @pl.loop(0, _CHUNKS // 2)
        def _pair(kk):
            a = 2 * kk
            start(a + 1, rows1, sem1)
            drain(a, rows0, sem0)
            reduce(a, rows0)
            start(jnp.minimum(a + 2, _CHUNKS - 1), rows0, sem0)
            drain(a + 1, rows1, sem1)
            reduce(a + 1, rows1)

        drain(_CHUNKS - 1, rows0, sem0)

        pltpu.sync_copy(out_v, out_hbm.at[pl.ds(sess0, _SESS_PER_W)])

    return k(table, idx)


def _finish_body(p_ref, w_ref, b_ref, out_ref):
    pooled = p_ref[...] * (1.0 / L)
    mm = jnp.dot(pooled, w_ref[...],
                 preferred_element_type=jnp.float32,
                 precision=jax.lax.Precision.HIGHEST)
    out_ref[...] = jnp.tanh(mm + b_ref[...])


def _finish(pooled_sum, sess_W, sess_b):
    return pl.pallas_call(
        _finish_body,
        out_shape=jax.ShapeDtypeStruct((B, TD), jnp.float32),
    )(pooled_sum, sess_W, sess_b.reshape(1, TD))


def kernel(sess2items, item_emb_table, item_features, feat_W, feat_b,
           sess_W, sess_b):
    idx = sess2items.astype(jnp.int32)
    table = _build_table(item_emb_table, item_features, feat_W, feat_b)
    pooled_sum = _gather_sum(table, idx)
    session_embedding = _finish(pooled_sum, sess_W, sess_b)
    return session_embedding, table


# speedup vs baseline: 1.8476x; 1.8476x over previous
import functools

import jax
import jax.numpy as jnp
from jax import lax
from jax.experimental import pallas as pl
from jax.experimental.pallas import tpu as pltpu
from jax.experimental.pallas import tpu_sc as plsc

V = 100000
D = 64
F = 128
B = 4096
L = 50
TD = 2 * D

_NC, _NS = 2, 16
_NW = _NC * _NS
_SESS_PER_W = B // _NW
_CH = 4
_CHUNKS = _SESS_PER_W // _CH
_LANES = 16
_NSEG = TD // _LANES


_ROW_BLK = 8192


def _table_body(embT_ref, feats_ref, w_ref, b_ref, out_ref):
    out_ref[:, :D] = embT_ref[...].T
    mm = jnp.dot(feats_ref[...], w_ref[...],
                 preferred_element_type=jnp.float32)
    out_ref[:, D:] = mm + b_ref[...]


def _build_table(item_emb_table, item_features, feat_W, feat_b):
    n_rows = item_emb_table.shape[0]
    grid = (n_rows + _ROW_BLK - 1) // _ROW_BLK
    return pl.pallas_call(
        _table_body,
        grid=(grid,),
        in_specs=[
            pl.BlockSpec((D, _ROW_BLK), lambda i: (0, i)),
            pl.BlockSpec((_ROW_BLK, F), lambda i: (i, 0)),
            pl.BlockSpec((F, D), lambda i: (0, 0)),
            pl.BlockSpec((1, D), lambda i: (0, 0)),
        ],
        out_specs=pl.BlockSpec((_ROW_BLK, TD), lambda i: (i, 0)),
        out_shape=jax.ShapeDtypeStruct((n_rows, TD), jnp.float32),
    )(item_emb_table.T, item_features, feat_W, feat_b.reshape(1, D))


def _gather_sum(table, idx):
    mesh = plsc.VectorSubcoreMesh(core_axis_name="c", subcore_axis_name="s")

    @functools.partial(
        pl.kernel,
        out_type=jax.ShapeDtypeStruct((B, TD), jnp.float32),
        mesh=mesh,
        scratch_types=[
            pltpu.VMEM((_SESS_PER_W, L), jnp.int32),
            pltpu.VMEM((_CH, L, TD), jnp.float32),
            pltpu.VMEM((_CH, L, TD), jnp.float32),
            pltpu.VMEM((_SESS_PER_W, TD), jnp.float32),
            pltpu.SemaphoreType.DMA,
            pltpu.SemaphoreType.DMA,
        ],
    )
    def k(table_hbm, idx_hbm, out_hbm, idx_all, rows0, rows1, out_v,
          sem0, sem1):
        wid = lax.axis_index("s") * _NC + lax.axis_index("c")
        sess0 = wid * _SESS_PER_W
        pltpu.sync_copy(idx_hbm.at[pl.ds(sess0, _SESS_PER_W)], idx_all)

        def start(j, rows_v, sem):
            for s in range(_CH):
                pltpu.async_copy(
                    table_hbm.at[idx_all.at[j * _CH + s]], rows_v.at[s], sem)

        def drain(j, rows_v, sem):
            for s in range(_CH):
                pltpu.make_async_copy(
                    table_hbm.at[idx_all.at[j * _CH + s]], rows_v.at[s],
                    sem).wait()

        def reduce(j, rows_v):
            for s in range(_CH):
                def body(l10, accs, s=s):
                    base = l10 * 10
                    for u in range(10):
                        accs = tuple(
                            acc + rows_v[s, base + u,
                                         pl.ds(c * _LANES, _LANES)]
                            for c, acc in enumerate(accs)
                        )
                    return accs
                accs = lax.fori_loop(
                    0, L // 10, body,
                    tuple(jnp.zeros((_LANES,), jnp.float32)
                          for _ in range(_NSEG)))
                for c in range(_NSEG):
                    out_v[j * _CH + s, pl.ds(c * _LANES, _LANES)] = accs[c]

        start(0, rows0, sem0)

        @pl.loop(0, _CHUNKS // 2)
        def _pair(kk):
            a = 2 * kk
            start(a + 1, rows1, sem1)
            drain(a, rows0, sem0)
            reduce(a, rows0)
            start(jnp.minimum(a + 2, _CHUNKS - 1), rows0, sem0)
            drain(a + 1, rows1, sem1)
            reduce(a + 1, rows1)

        drain(_CHUNKS - 1, rows0, sem0)

        pltpu.sync_copy(out_v, out_hbm.at[pl.ds(sess0, _SESS_PER_W)])

    return k(table, idx)


def _finish_body(p_ref, w_ref, b_ref, out_ref):
    pooled = p_ref[...] * (1.0 / L)
    mm = jnp.dot(pooled, w_ref[...],
                 preferred_element_type=jnp.float32,
                 precision=jax.lax.Precision.HIGHEST)
    out_ref[...] = jnp.tanh(mm + b_ref[...])


def _finish(pooled_sum, sess_W, sess_b):
    return pl.pallas_call(
        _finish_body,
        out_shape=jax.ShapeDtypeStruct((B, TD), jnp.float32),
    )(pooled_sum, sess_W, sess_b.reshape(1, TD))


def kernel(sess2items, item_emb_table, item_features, feat_W, feat_b,
           sess_W, sess_b):
    idx = sess2items.astype(jnp.int32)
    table = _build_table(item_emb_table, item_features, feat_W, feat_b)
    pooled_sum = _gather_sum(table, idx)
    session_embedding = _finish(pooled_sum, sess_W, sess_b)
    return session_embedding, table


# speedup vs baseline: 1.8624x; 1.0080x over previous
import functools

import jax
import jax.numpy as jnp
from jax import lax
from jax.experimental import pallas as pl
from jax.experimental.pallas import tpu as pltpu
from jax.experimental.pallas import tpu_sc as plsc

V = 100000
D = 64
F = 128
B = 4096
L = 50
TD = 2 * D

_NC, _NS = 2, 16
_NW = _NC * _NS
_SESS_PER_W = B // _NW
_CH = 4
_CHUNKS = _SESS_PER_W // _CH
_LANES = 16
_NSEG = TD // _LANES


_ROW_BLK = 8192


def _table_body(embT_ref, feats_ref, w_ref, b_ref, out_ref):
    out_ref[:, :D] = embT_ref[...].T
    mm = jnp.dot(feats_ref[...], w_ref[...],
                 preferred_element_type=jnp.float32)
    out_ref[:, D:] = mm + b_ref[...]


def _build_table(item_emb_table, item_features, feat_W, feat_b):
    n_rows = item_emb_table.shape[0]
    grid = (n_rows + _ROW_BLK - 1) // _ROW_BLK
    return pl.pallas_call(
        _table_body,
        grid=(grid,),
        in_specs=[
            pl.BlockSpec((D, _ROW_BLK), lambda i: (0, i)),
            pl.BlockSpec((_ROW_BLK, F), lambda i: (i, 0)),
            pl.BlockSpec((F, D), lambda i: (0, 0)),
            pl.BlockSpec((1, D), lambda i: (0, 0)),
        ],
        out_specs=pl.BlockSpec((_ROW_BLK, TD), lambda i: (i, 0)),
        out_shape=jax.ShapeDtypeStruct((n_rows, TD), jnp.float32),
    )(item_emb_table.T, item_features, feat_W, feat_b.reshape(1, D))


def _gather_sum(table, idx):
    mesh = plsc.VectorSubcoreMesh(core_axis_name="c", subcore_axis_name="s")

    @functools.partial(
        pl.kernel,
        out_type=jax.ShapeDtypeStruct((B, TD), jnp.float32),
        mesh=mesh,
        scratch_types=[
            pltpu.VMEM((_SESS_PER_W, L), jnp.int32),
            pltpu.VMEM((_CH, L, TD), jnp.float32),
            pltpu.VMEM((_CH, L, TD), jnp.float32),
            pltpu.VMEM((_SESS_PER_W, TD), jnp.float32),
            pltpu.SemaphoreType.DMA,
            pltpu.SemaphoreType.DMA,
        ],
    )
    def k(table_hbm, idx_hbm, out_hbm, idx_all, rows0, rows1, out_v,
          sem0, sem1):
        wid = lax.axis_index("s") * _NC + lax.axis_index("c")
        sess0 = wid * _SESS_PER_W
        pltpu.sync_copy(idx_hbm.at[pl.ds(sess0, _SESS_PER_W)], idx_all)

        def start(j, rows_v, sem):
            for s in range(_CH):
                pltpu.async_copy(
                    table_hbm.at[idx_all.at[j * _CH + s]], rows_v.at[s], sem)

        def drain(j, rows_v, sem):
            for s in range(_CH):
                pltpu.make_async_copy(
                    table_hbm.at[idx_all.at[j * _CH + s]], rows_v.at[s],
                    sem).wait()

        def reduce(j, rows_v):
            for s in range(_CH):
                def body(l, accs, s=s):
                    return tuple(
                        acc + rows_v[s, l, pl.ds(c * _LANES, _LANES)]
                        for c, acc in enumerate(accs)
                    )
                accs = plsc.parallel_loop(
                    0, L, unroll=10,
                    carry=tuple(jnp.zeros((_LANES,), jnp.float32)
                                for _ in range(_NSEG)))(body)
                for c in range(_NSEG):
                    out_v[j * _CH + s, pl.ds(c * _LANES, _LANES)] = accs[c]

        start(0, rows0, sem0)

        @pl.loop(0, _CHUNKS // 2)
        def _pair(kk):
            a = 2 * kk
            start(a + 1, rows1, sem1)
            drain(a, rows0, sem0)
            reduce(a, rows0)
            start(jnp.minimum(a + 2, _CHUNKS - 1), rows0, sem0)
            drain(a + 1, rows1, sem1)
            reduce(a + 1, rows1)

        drain(_CHUNKS - 1, rows0, sem0)

        pltpu.sync_copy(out_v, out_hbm.at[pl.ds(sess0, _SESS_PER_W)])

    return k(table, idx)


def _finish_body(p_ref, w_ref, b_ref, out_ref):
    pooled = p_ref[...] * (1.0 / L)
    mm = jnp.dot(pooled, w_ref[...],
                 preferred_element_type=jnp.float32,
                 precision=jax.lax.Precision.HIGHEST)
    out_ref[...] = jnp.tanh(mm + b_ref[...])


def _finish(pooled_sum, sess_W, sess_b):
    return pl.pallas_call(
        _finish_body,
        out_shape=jax.ShapeDtypeStruct((B, TD), jnp.float32),
    )(pooled_sum, sess_W, sess_b.reshape(1, TD))


def kernel(sess2items, item_emb_table, item_features, feat_W, feat_b,
           sess_W, sess_b):
    idx = sess2items.astype(jnp.int32)
    table = _build_table(item_emb_table, item_features, feat_W, feat_b)
    pooled_sum = _gather_sum(table, idx)
    session_embedding = _finish(pooled_sum, sess_W, sess_b)
    return session_embedding, table


# speedup vs baseline: 1.8670x; 1.0025x over previous
import functools

import jax
import jax.numpy as jnp
from jax import lax
from jax.experimental import pallas as pl
from jax.experimental.pallas import tpu as pltpu
from jax.experimental.pallas import tpu_sc as plsc

V = 100000
D = 64
F = 128
B = 4096
L = 50
TD = 2 * D

_NC, _NS = 2, 16
_NW = _NC * _NS
_SESS_PER_W = B // _NW
_CH = 4
_CHUNKS = _SESS_PER_W // _CH
_LANES = 16
_NSEG = TD // _LANES


_ROW_BLK = 16384


def _table_body(embT_ref, feats_ref, w_ref, b_ref, out_ref):
    out_ref[:, :D] = embT_ref[...].T
    mm = jnp.dot(feats_ref[...], w_ref[...],
                 preferred_element_type=jnp.float32)
    out_ref[:, D:] = mm + b_ref[...]


def _build_table(item_emb_table, item_features, feat_W, feat_b):
    n_rows = item_emb_table.shape[0]
    grid = (n_rows + _ROW_BLK - 1) // _ROW_BLK
    return pl.pallas_call(
        _table_body,
        grid=(grid,),
        in_specs=[
            pl.BlockSpec((D, _ROW_BLK), lambda i: (0, i)),
            pl.BlockSpec((_ROW_BLK, F), lambda i: (i, 0)),
            pl.BlockSpec((F, D), lambda i: (0, 0)),
            pl.BlockSpec((1, D), lambda i: (0, 0)),
        ],
        out_specs=pl.BlockSpec((_ROW_BLK, TD), lambda i: (i, 0)),
        out_shape=jax.ShapeDtypeStruct((n_rows, TD), jnp.float32),
    )(item_emb_table.T, item_features, feat_W, feat_b.reshape(1, D))


def _gather_sum(table, idx):
    mesh = plsc.VectorSubcoreMesh(core_axis_name="c", subcore_axis_name="s")

    @functools.partial(
        pl.kernel,
        out_type=jax.ShapeDtypeStruct((B, TD), jnp.float32),
        mesh=mesh,
        scratch_types=[
            pltpu.VMEM((_SESS_PER_W, L), jnp.int32),
            pltpu.VMEM((_CH, L, TD), jnp.float32),
            pltpu.VMEM((_CH, L, TD), jnp.float32),
            pltpu.VMEM((_SESS_PER_W, TD), jnp.float32),
            pltpu.SemaphoreType.DMA,
            pltpu.SemaphoreType.DMA,
        ],
    )
    def k(table_hbm, idx_hbm, out_hbm, idx_all, rows0, rows1, out_v,
          sem0, sem1):
        wid = lax.axis_index("s") * _NC + lax.axis_index("c")
        sess0 = wid * _SESS_PER_W
        pltpu.sync_copy(idx_hbm.at[pl.ds(sess0, _SESS_PER_W)], idx_all)

        def start(j, rows_v, sem):
            for s in range(_CH):
                pltpu.async_copy(
                    table_hbm.at[idx_all.at[j * _CH + s]], rows_v.at[s], sem)

        def drain(j, rows_v, sem):
            for s in range(_CH):
                pltpu.make_async_copy(
                    table_hbm.at[idx_all.at[j * _CH + s]], rows_v.at[s],
                    sem).wait()

        def reduce(j, rows_v):
            for s in range(_CH):
                def body(l, accs, s=s):
                    return tuple(
                        acc + rows_v[s, l, pl.ds(c * _LANES, _LANES)]
                        for c, acc in enumerate(accs)
                    )
                accs = plsc.parallel_loop(
                    0, L, unroll=10,
                    carry=tuple(jnp.zeros((_LANES,), jnp.float32)
                                for _ in range(_NSEG)))(body)
                for c in range(_NSEG):
                    out_v[j * _CH + s, pl.ds(c * _LANES, _LANES)] = accs[c]

        start(0, rows0, sem0)

        @pl.loop(0, _CHUNKS // 2)
        def _pair(kk):
            a = 2 * kk
            start(a + 1, rows1, sem1)
            drain(a, rows0, sem0)
            reduce(a, rows0)
            start(jnp.minimum(a + 2, _CHUNKS - 1), rows0, sem0)
            drain(a + 1, rows1, sem1)
            reduce(a + 1, rows1)

        drain(_CHUNKS - 1, rows0, sem0)

        pltpu.sync_copy(out_v, out_hbm.at[pl.ds(sess0, _SESS_PER_W)])

    return k(table, idx)


def _finish_body(p_ref, w_ref, b_ref, out_ref):
    pooled = p_ref[...] * (1.0 / L)
    mm = jnp.dot(pooled, w_ref[...],
                 preferred_element_type=jnp.float32,
                 precision=jax.lax.Precision.HIGHEST)
    out_ref[...] = jnp.tanh(mm + b_ref[...])


def _finish(pooled_sum, sess_W, sess_b):
    return pl.pallas_call(
        _finish_body,
        out_shape=jax.ShapeDtypeStruct((B, TD), jnp.float32),
    )(pooled_sum, sess_W, sess_b.reshape(1, TD))


def kernel(sess2items, item_emb_table, item_features, feat_W, feat_b,
           sess_W, sess_b):
    idx = sess2items.astype(jnp.int32)
    table = _build_table(item_emb_table, item_features, feat_W, feat_b)
    pooled_sum = _gather_sum(table, idx)
    session_embedding = _finish(pooled_sum, sess_W, sess_b)
    return session_embedding, table
